# baseline (speedup 1.0000x reference)
import functools

import jax
import jax.numpy as jnp
from jax import lax
from jax.experimental import pallas as pl
from jax.experimental.pallas import tpu as pltpu
from jax.experimental.pallas import tpu_sc as plsc

EDIM = 32
KCODES = 8192
NTOK = 8192
NBATCH = 8
TOK_PER_BATCH = NTOK // NBATCH
CCOEF = 0.25


BN1 = 256
NI1 = NTOK // BN1


def _dist_body(x_ref, wt_ref, d_ref, idx_ref):
    x = x_ref[...]
    wt = wt_ref[...]
    a = jnp.sum(x * x, axis=1, keepdims=True)
    b = jnp.sum(wt * wt, axis=0, keepdims=True)
    m2 = lax.dot_general(x + x, wt, (((1,), (0,)), ((), ())),
                         preferred_element_type=jnp.float32)
    d = (a + b) - m2
    d_ref[...] = d
    mn = jnp.min(d, axis=1, keepdims=True)
    iota = lax.broadcasted_iota(jnp.int32, (1, KCODES), 1)
    big = jnp.int32(2**31 - 1)
    idx_ref[...] = jnp.min(jnp.where(d == mn, iota, big),
                           axis=1, keepdims=True)


def _distances_argmin(flat, wt):
    return pl.pallas_call(
        _dist_body,
        grid=(NI1,),
        in_specs=[
            pl.BlockSpec((BN1, EDIM), lambda i: (i, 0)),
            pl.BlockSpec((EDIM, KCODES), lambda i: (0, 0)),
        ],
        out_specs=[
            pl.BlockSpec((BN1, KCODES), lambda i: (i, 0)),
            pl.BlockSpec((BN1, 1), lambda i: (i, 0)),
        ],
        out_shape=[
            jax.ShapeDtypeStruct((NTOK, KCODES), jnp.float32),
            jax.ShapeDtypeStruct((NTOK, 1), jnp.int32),
        ],
    )(flat, wt)


GATHER_W = 128
GATHER_CHUNK = 128


def _sc_gather(idx_flat, table128):
    info = plsc.get_sparse_core_info()
    nw = info.num_cores * info.num_subcores
    bpw = NTOK // nw
    nchunk = bpw // GATHER_CHUNK
    mesh = plsc.VectorSubcoreMesh(core_axis_name="c", subcore_axis_name="s")

    @functools.partial(
        pl.kernel,
        mesh=mesh,
        out_type=jax.ShapeDtypeStruct((NTOK, GATHER_W), jnp.float32),
        scratch_types=[
            pltpu.VMEM((bpw,), jnp.int32),
            pltpu.VMEM((bpw, GATHER_W), jnp.float32),
            pltpu.SemaphoreType.DMA,
        ],
    )
    def gather_k(idx_hbm, table_hbm, out_hbm, idx_v, rows_v, sem):
        wid = lax.axis_index("s") * info.num_cores + lax.axis_index("c")
        base = wid * bpw
        pltpu.sync_copy(idx_hbm.at[pl.ds(base, bpw)], idx_v)
        descs = []
        for c in range(nchunk):
            off = c * GATHER_CHUNK
            descs.append(pltpu.async_copy(
                table_hbm.at[idx_v.at[pl.ds(off, GATHER_CHUNK)]],
                rows_v.at[pl.ds(off, GATHER_CHUNK)], sem))
        for dsc in descs:
            dsc.wait()
        pltpu.sync_copy(rows_v, out_hbm.at[pl.ds(base, bpw)])

    return gather_k(idx_flat, table128)


BN2 = TOK_PER_BATCH
BK2 = 1024
NI2 = NTOK // BN2
NJ2 = KCODES // BK2
INV_N = 1.0 / NTOK
INV_ELEMS = 1.0 / (TOK_PER_BATCH * EDIM)


def _enc_body(idx_ref, x_ref, q_ref, enc_ref, perp_ref, qst_ref, loss_ref,
              counts_s, ent_s):
    i = pl.program_id(0)
    j = pl.program_id(1)
    col0 = pl.multiple_of(j * BK2, BK2)

    idxv = idx_ref[...]
    iota = lax.broadcasted_iota(jnp.int32, (1, BK2), 1) + col0
    onehot = (iota == idxv).astype(jnp.float32)
    enc_ref[...] = onehot

    colsum = jnp.sum(onehot, axis=0, keepdims=True)

    @pl.when(j == 0)
    def _():
        x = x_ref[...]
        q = q_ref[:, :EDIM]
        t = q - x
        qst_ref[...] = x + t
        s = jnp.sum(t * t, axis=(0, 1), keepdims=True) * INV_ELEMS
        loss_ref[0] = s + CCOEF * s

    @pl.when(i == 0)
    def _():
        counts_s[:, pl.ds(col0, BK2)] = colsum

    @pl.when(i > 0)
    def _():
        counts_s[:, pl.ds(col0, BK2)] += colsum

    @pl.when(i == NI2 - 1)
    def _():
        p = counts_s[:, pl.ds(col0, BK2)] * INV_N
        part = jnp.sum(p * jnp.log(p + 1e-10), axis=1, keepdims=True)

        @pl.when(j == 0)
        def _():
            ent_s[...] = part

        @pl.when(j > 0)
        def _():
            ent_s[...] += part

        @pl.when(j == NJ2 - 1)
        def _():
            perp_ref[...] = jnp.exp(-ent_s[...])


def _encodings_stats(idx_col, flat, quant128):
    return pl.pallas_call(
        _enc_body,
        grid=(NI2, NJ2),
        in_specs=[
            pl.BlockSpec((BN2, 1), lambda i, j: (i, 0)),
            pl.BlockSpec((BN2, EDIM), lambda i, j: (i, 0)),
            pl.BlockSpec((BN2, GATHER_W), lambda i, j: (i, 0)),
        ],
        out_specs=[
            pl.BlockSpec((BN2, BK2), lambda i, j: (i, j)),
            pl.BlockSpec((1, 1), lambda i, j: (0, 0)),
            pl.BlockSpec((BN2, EDIM), lambda i, j: (i, 0)),
            pl.BlockSpec((1, 1, 1), lambda i, j: (i, 0, 0)),
        ],
        out_shape=[
            jax.ShapeDtypeStruct((NTOK, KCODES), jnp.float32),
            jax.ShapeDtypeStruct((1, 1), jnp.float32),
            jax.ShapeDtypeStruct((NTOK, EDIM), jnp.float32),
            jax.ShapeDtypeStruct((NBATCH, 1, 1), jnp.float32),
        ],
        scratch_shapes=[
            pltpu.VMEM((1, KCODES), jnp.float32),
            pltpu.VMEM((1, 1), jnp.float32),
        ],
    )(idx_col, flat, quant128)


def kernel(inputs, W):
    x = jnp.transpose(inputs, (0, 2, 3, 1))
    flat = x.reshape(NTOK, EDIM)
    wt = W.T

    distances, idx_col = _distances_argmin(flat, wt)
    table128 = jnp.pad(W, ((0, 0), (0, GATHER_W - EDIM)))
    quant128 = _sc_gather(idx_col.reshape(NTOK), table128)
    encodings, perp2, qst, loss3 = _encodings_stats(idx_col, flat, quant128)

    quantized_out = jnp.transpose(qst.reshape(NBATCH, 32, 32, EDIM),
                                  (0, 3, 1, 2))
    loss = loss3.reshape(NBATCH)
    perplexity = perp2.reshape(())
    enc_idx = idx_col.reshape(NBATCH, TOK_PER_BATCH)
    return (quantized_out, loss, perplexity, encodings, enc_idx, distances)


# speedup vs baseline: 1.0069x; 1.0069x over previous
import functools

import jax
import jax.numpy as jnp
from jax import lax
from jax.experimental import pallas as pl
from jax.experimental.pallas import tpu as pltpu
from jax.experimental.pallas import tpu_sc as plsc

EDIM = 32
KCODES = 8192
NTOK = 8192
NBATCH = 8
TOK_PER_BATCH = NTOK // NBATCH
CCOEF = 0.25


BN1 = 512
NI1 = NTOK // BN1


def _dist_body(x_ref, wt_ref, d_ref, idx_ref):
    x = x_ref[...]
    wt = wt_ref[...]
    a = jnp.sum(x * x, axis=1, keepdims=True)
    b = jnp.sum(wt * wt, axis=0, keepdims=True)
    m2 = lax.dot_general(x + x, wt, (((1,), (0,)), ((), ())),
                         preferred_element_type=jnp.float32)
    d = (a + b) - m2
    d_ref[...] = d
    mn = jnp.min(d, axis=1, keepdims=True)
    iota = lax.broadcasted_iota(jnp.int32, (1, KCODES), 1)
    big = jnp.int32(2**31 - 1)
    idx_ref[...] = jnp.min(jnp.where(d == mn, iota, big),
                           axis=1, keepdims=True)


def _distances_argmin(flat, wt):
    return pl.pallas_call(
        _dist_body,
        grid=(NI1,),
        in_specs=[
            pl.BlockSpec((BN1, EDIM), lambda i: (i, 0)),
            pl.BlockSpec((EDIM, KCODES), lambda i: (0, 0)),
        ],
        out_specs=[
            pl.BlockSpec((BN1, KCODES), lambda i: (i, 0)),
            pl.BlockSpec((BN1, 1), lambda i: (i, 0)),
        ],
        out_shape=[
            jax.ShapeDtypeStruct((NTOK, KCODES), jnp.float32),
            jax.ShapeDtypeStruct((NTOK, 1), jnp.int32),
        ],
    )(flat, wt)


GATHER_W = 128
GATHER_CHUNK = 128


def _sc_gather(idx_flat, table128):
    info = plsc.get_sparse_core_info()
    nw = info.num_cores * info.num_subcores
    bpw = NTOK // nw
    nchunk = bpw // GATHER_CHUNK
    mesh = plsc.VectorSubcoreMesh(core_axis_name="c", subcore_axis_name="s")

    @functools.partial(
        pl.kernel,
        mesh=mesh,
        out_type=jax.ShapeDtypeStruct((NTOK, GATHER_W), jnp.float32),
        scratch_types=[
            pltpu.VMEM((bpw,), jnp.int32),
            pltpu.VMEM((bpw, GATHER_W), jnp.float32),
            pltpu.SemaphoreType.DMA,
        ],
    )
    def gather_k(idx_hbm, table_hbm, out_hbm, idx_v, rows_v, sem):
        wid = lax.axis_index("s") * info.num_cores + lax.axis_index("c")
        base = wid * bpw
        pltpu.sync_copy(idx_hbm.at[pl.ds(base, bpw)], idx_v)
        descs = []
        for c in range(nchunk):
            off = c * GATHER_CHUNK
            descs.append(pltpu.async_copy(
                table_hbm.at[idx_v.at[pl.ds(off, GATHER_CHUNK)]],
                rows_v.at[pl.ds(off, GATHER_CHUNK)], sem))
        for dsc in descs:
            dsc.wait()
        pltpu.sync_copy(rows_v, out_hbm.at[pl.ds(base, bpw)])

    return gather_k(idx_flat, table128)


BN2 = TOK_PER_BATCH
BK2 = 1024
NI2 = NTOK // BN2
NJ2 = KCODES // BK2
INV_N = 1.0 / NTOK
INV_ELEMS = 1.0 / (TOK_PER_BATCH * EDIM)


def _enc_body(idx_ref, x_ref, q_ref, enc_ref, perp_ref, qst_ref, loss_ref,
              counts_s, ent_s):
    i = pl.program_id(0)
    j = pl.program_id(1)
    col0 = pl.multiple_of(j * BK2, BK2)

    idxv = idx_ref[...]
    iota = lax.broadcasted_iota(jnp.int32, (1, BK2), 1) + col0
    onehot = (iota == idxv).astype(jnp.float32)
    enc_ref[...] = onehot

    colsum = jnp.sum(onehot, axis=0, keepdims=True)

    @pl.when(j == 0)
    def _():
        x = x_ref[...]
        q = q_ref[:, :EDIM]
        t = q - x
        qst_ref[...] = x + t
        s = jnp.sum(t * t, axis=(0, 1), keepdims=True) * INV_ELEMS
        loss_ref[0] = s + CCOEF * s

    @pl.when(i == 0)
    def _():
        counts_s[:, pl.ds(col0, BK2)] = colsum

    @pl.when(i > 0)
    def _():
        counts_s[:, pl.ds(col0, BK2)] += colsum

    @pl.when(i == NI2 - 1)
    def _():
        p = counts_s[:, pl.ds(col0, BK2)] * INV_N
        part = jnp.sum(p * jnp.log(p + 1e-10), axis=1, keepdims=True)

        @pl.when(j == 0)
        def _():
            ent_s[...] = part

        @pl.when(j > 0)
        def _():
            ent_s[...] += part

        @pl.when(j == NJ2 - 1)
        def _():
            perp_ref[...] = jnp.exp(-ent_s[...])


def _encodings_stats(idx_col, flat, quant128):
    return pl.pallas_call(
        _enc_body,
        grid=(NI2, NJ2),
        in_specs=[
            pl.BlockSpec((BN2, 1), lambda i, j: (i, 0)),
            pl.BlockSpec((BN2, EDIM), lambda i, j: (i, 0)),
            pl.BlockSpec((BN2, GATHER_W), lambda i, j: (i, 0)),
        ],
        out_specs=[
            pl.BlockSpec((BN2, BK2), lambda i, j: (i, j)),
            pl.BlockSpec((1, 1), lambda i, j: (0, 0)),
            pl.BlockSpec((BN2, EDIM), lambda i, j: (i, 0)),
            pl.BlockSpec((1, 1, 1), lambda i, j: (i, 0, 0)),
        ],
        out_shape=[
            jax.ShapeDtypeStruct((NTOK, KCODES), jnp.float32),
            jax.ShapeDtypeStruct((1, 1), jnp.float32),
            jax.ShapeDtypeStruct((NTOK, EDIM), jnp.float32),
            jax.ShapeDtypeStruct((NBATCH, 1, 1), jnp.float32),
        ],
        scratch_shapes=[
            pltpu.VMEM((1, KCODES), jnp.float32),
            pltpu.VMEM((1, 1), jnp.float32),
        ],
    )(idx_col, flat, quant128)


def kernel(inputs, W):
    x = jnp.transpose(inputs, (0, 2, 3, 1))
    flat = x.reshape(NTOK, EDIM)
    wt = W.T

    distances, idx_col = _distances_argmin(flat, wt)
    table128 = jnp.pad(W, ((0, 0), (0, GATHER_W - EDIM)))
    quant128 = _sc_gather(idx_col.reshape(NTOK), table128)
    encodings, perp2, qst, loss3 = _encodings_stats(idx_col, flat, quant128)

    quantized_out = jnp.transpose(qst.reshape(NBATCH, 32, 32, EDIM),
                                  (0, 3, 1, 2))
    loss = loss3.reshape(NBATCH)
    perplexity = perp2.reshape(())
    enc_idx = idx_col.reshape(NBATCH, TOK_PER_BATCH)
    return (quantized_out, loss, perplexity, encodings, enc_idx, distances)


# speedup vs baseline: 1.0764x; 1.0691x over previous
import functools

import jax
import jax.numpy as jnp
from jax import lax
from jax.experimental import pallas as pl
from jax.experimental.pallas import tpu as pltpu
from jax.experimental.pallas import tpu_sc as plsc

EDIM = 32
KCODES = 8192
NTOK = 8192
NBATCH = 8
TOK_PER_BATCH = NTOK // NBATCH
CCOEF = 0.25


BN1 = 512
NI1 = NTOK // BN1


def _dist_body(x_ref, wt_ref, d_ref, idx_ref):
    x = x_ref[...]
    wt = wt_ref[...]
    a = jnp.sum(x * x, axis=1, keepdims=True)
    b = jnp.sum(wt * wt, axis=0, keepdims=True)
    m2 = lax.dot_general(x + x, wt, (((1,), (0,)), ((), ())),
                         preferred_element_type=jnp.float32)
    d = (a + b) - m2
    d_ref[...] = d
    mn = jnp.min(d, axis=1, keepdims=True)
    iota = lax.broadcasted_iota(jnp.int32, (1, KCODES), 1)
    big = jnp.int32(2**31 - 1)
    idx_ref[...] = jnp.min(jnp.where(d == mn, iota, big),
                           axis=1, keepdims=True)


def _distances_argmin(flat, wt):
    return pl.pallas_call(
        _dist_body,
        grid=(NI1,),
        in_specs=[
            pl.BlockSpec((BN1, EDIM), lambda i: (i, 0)),
            pl.BlockSpec((EDIM, KCODES), lambda i: (0, 0)),
        ],
        out_specs=[
            pl.BlockSpec((BN1, KCODES), lambda i: (i, 0)),
            pl.BlockSpec((BN1, 1), lambda i: (i, 0)),
        ],
        out_shape=[
            jax.ShapeDtypeStruct((NTOK, KCODES), jnp.float32),
            jax.ShapeDtypeStruct((NTOK, 1), jnp.int32),
        ],
    )(flat, wt)


GATHER_W = 128
GATHER_CHUNK = 128


def _sc_gather(idx_flat, table128):
    info = plsc.get_sparse_core_info()
    nw = info.num_cores * info.num_subcores
    bpw = NTOK // nw
    nchunk = bpw // GATHER_CHUNK
    mesh = plsc.VectorSubcoreMesh(core_axis_name="c", subcore_axis_name="s")

    @functools.partial(
        pl.kernel,
        mesh=mesh,
        out_type=jax.ShapeDtypeStruct((NTOK, GATHER_W), jnp.float32),
        scratch_types=[
            pltpu.VMEM((bpw,), jnp.int32),
            pltpu.VMEM((bpw, GATHER_W), jnp.float32),
            pltpu.SemaphoreType.DMA,
        ],
    )
    def gather_k(idx_hbm, table_hbm, out_hbm, idx_v, rows_v, sem):
        wid = lax.axis_index("s") * info.num_cores + lax.axis_index("c")
        base = wid * bpw
        pltpu.sync_copy(idx_hbm.at[pl.ds(base, bpw)], idx_v)
        descs = []
        for c in range(nchunk):
            off = c * GATHER_CHUNK
            descs.append(pltpu.async_copy(
                table_hbm.at[idx_v.at[pl.ds(off, GATHER_CHUNK)]],
                rows_v.at[pl.ds(off, GATHER_CHUNK)], sem))
        for dsc in descs:
            dsc.wait()
        pltpu.sync_copy(rows_v, out_hbm.at[pl.ds(base, bpw)])

    return gather_k(idx_flat, table128)


BN2 = TOK_PER_BATCH
BK2 = 2048
NI2 = NTOK // BN2
NJ2 = KCODES // BK2
INV_N = 1.0 / NTOK
INV_ELEMS = 1.0 / (TOK_PER_BATCH * EDIM)


def _enc_body(idx_ref, enc_ref, perp_ref, counts_s, ent_s):
    i = pl.program_id(0)
    j = pl.program_id(1)
    col0 = pl.multiple_of(j * BK2, BK2)

    idxv = idx_ref[...]
    iota = lax.broadcasted_iota(jnp.int32, (1, BK2), 1) + col0
    onehot = (iota == idxv).astype(jnp.float32)
    enc_ref[...] = onehot

    colsum = jnp.sum(onehot, axis=0, keepdims=True)

    @pl.when(i == 0)
    def _():
        counts_s[:, pl.ds(col0, BK2)] = colsum

    @pl.when(i > 0)
    def _():
        counts_s[:, pl.ds(col0, BK2)] += colsum

    @pl.when(i == NI2 - 1)
    def _():
        p = counts_s[:, pl.ds(col0, BK2)] * INV_N
        part = jnp.sum(p * jnp.log(p + 1e-10), axis=1, keepdims=True)

        @pl.when(j == 0)
        def _():
            ent_s[...] = part

        @pl.when(j > 0)
        def _():
            ent_s[...] += part

        @pl.when(j == NJ2 - 1)
        def _():
            perp_ref[...] = jnp.exp(-ent_s[...])


def _encodings_stats(idx_col):
    return pl.pallas_call(
        _enc_body,
        grid=(NI2, NJ2),
        in_specs=[
            pl.BlockSpec((BN2, 1), lambda i, j: (i, 0)),
        ],
        out_specs=[
            pl.BlockSpec((BN2, BK2), lambda i, j: (i, j)),
            pl.BlockSpec((1, 1), lambda i, j: (0, 0)),
        ],
        out_shape=[
            jax.ShapeDtypeStruct((NTOK, KCODES), jnp.float32),
            jax.ShapeDtypeStruct((1, 1), jnp.float32),
        ],
        scratch_shapes=[
            pltpu.VMEM((1, KCODES), jnp.float32),
            pltpu.VMEM((1, 1), jnp.float32),
        ],
    )(idx_col)


def _qst_body(x_ref, q_ref, qst_ref, loss_ref):
    x = x_ref[...]
    q = q_ref[:, :EDIM]
    t = q - x
    qst_ref[...] = x + t
    tsq = t * t
    parts = [jnp.sum(tsq[i * TOK_PER_BATCH:(i + 1) * TOK_PER_BATCH, :],
                     axis=(0, 1), keepdims=True) for i in range(NBATCH)]
    s = jnp.concatenate(parts, axis=0) * INV_ELEMS
    loss_ref[...] = s + CCOEF * s


def _qst_loss(flat, quant128):
    return pl.pallas_call(
        _qst_body,
        in_specs=[
            pl.BlockSpec((NTOK, EDIM), lambda: (0, 0)),
            pl.BlockSpec((NTOK, GATHER_W), lambda: (0, 0)),
        ],
        out_specs=[
            pl.BlockSpec((NTOK, EDIM), lambda: (0, 0)),
            pl.BlockSpec((NBATCH, 1), lambda: (0, 0)),
        ],
        out_shape=[
            jax.ShapeDtypeStruct((NTOK, EDIM), jnp.float32),
            jax.ShapeDtypeStruct((NBATCH, 1), jnp.float32),
        ],
    )(flat, quant128)


def kernel(inputs, W):
    x = jnp.transpose(inputs, (0, 2, 3, 1))
    flat = x.reshape(NTOK, EDIM)
    wt = W.T

    distances, idx_col = _distances_argmin(flat, wt)
    table128 = jnp.pad(W, ((0, 0), (0, GATHER_W - EDIM)))
    encodings, perp2 = _encodings_stats(idx_col)
    quant128 = _sc_gather(idx_col.reshape(NTOK), table128)
    qst, loss3 = _qst_loss(flat, quant128)

    quantized_out = jnp.transpose(qst.reshape(NBATCH, 32, 32, EDIM),
                                  (0, 3, 1, 2))
    loss = loss3.reshape(NBATCH)
    perplexity = perp2.reshape(())
    enc_idx = idx_col.reshape(NBATCH, TOK_PER_BATCH)
    return (quantized_out, loss, perplexity, encodings, enc_idx, distances)


# speedup vs baseline: 1.0765x; 1.0001x over previous
import functools

import jax
import jax.numpy as jnp
from jax import lax
from jax.experimental import pallas as pl
from jax.experimental.pallas import tpu as pltpu
from jax.experimental.pallas import tpu_sc as plsc

EDIM = 32
KCODES = 8192
NTOK = 8192
NBATCH = 8
TOK_PER_BATCH = NTOK // NBATCH
CCOEF = 0.25


BN1 = 512
NI1 = NTOK // BN1


def _dist_body(x_ref, wt_ref, d_ref, idx_ref):
    x = x_ref[...]
    wt = wt_ref[...]
    a = jnp.sum(x * x, axis=1, keepdims=True)
    b = jnp.sum(wt * wt, axis=0, keepdims=True)
    m2 = lax.dot_general(x + x, wt, (((1,), (0,)), ((), ())),
                         preferred_element_type=jnp.float32)
    d = (a + b) - m2
    d_ref[...] = d
    mn = jnp.min(d, axis=1, keepdims=True)
    iota = lax.broadcasted_iota(jnp.int32, (1, KCODES), 1)
    big = jnp.int32(2**31 - 1)
    idx_ref[...] = jnp.min(jnp.where(d == mn, iota, big),
                           axis=1, keepdims=True)


def _distances_argmin(flat, wt):
    return pl.pallas_call(
        _dist_body,
        grid=(NI1,),
        in_specs=[
            pl.BlockSpec((BN1, EDIM), lambda i: (i, 0)),
            pl.BlockSpec((EDIM, KCODES), lambda i: (0, 0)),
        ],
        out_specs=[
            pl.BlockSpec((BN1, KCODES), lambda i: (i, 0)),
            pl.BlockSpec((BN1, 1), lambda i: (i, 0)),
        ],
        out_shape=[
            jax.ShapeDtypeStruct((NTOK, KCODES), jnp.float32),
            jax.ShapeDtypeStruct((NTOK, 1), jnp.int32),
        ],
    )(flat, wt)


GATHER_W = 128
GATHER_CHUNK = 128


def _sc_gather(idx_flat, table128):
    info = plsc.get_sparse_core_info()
    nw = info.num_cores * info.num_subcores
    bpw = NTOK // nw
    nchunk = bpw // GATHER_CHUNK
    mesh = plsc.VectorSubcoreMesh(core_axis_name="c", subcore_axis_name="s")

    @functools.partial(
        pl.kernel,
        mesh=mesh,
        out_type=jax.ShapeDtypeStruct((NTOK, GATHER_W), jnp.float32),
        scratch_types=[
            pltpu.VMEM((bpw,), jnp.int32),
            pltpu.VMEM((bpw, GATHER_W), jnp.float32),
            pltpu.SemaphoreType.DMA,
        ],
    )
    def gather_k(idx_hbm, table_hbm, out_hbm, idx_v, rows_v, sem):
        wid = lax.axis_index("s") * info.num_cores + lax.axis_index("c")
        base = wid * bpw
        pltpu.sync_copy(idx_hbm.at[pl.ds(base, bpw)], idx_v)
        descs = []
        for c in range(nchunk):
            off = c * GATHER_CHUNK
            descs.append(pltpu.async_copy(
                table_hbm.at[idx_v.at[pl.ds(off, GATHER_CHUNK)]],
                rows_v.at[pl.ds(off, GATHER_CHUNK)], sem))
        for dsc in descs:
            dsc.wait()
        pltpu.sync_copy(rows_v, out_hbm.at[pl.ds(base, bpw)])

    return gather_k(idx_flat, table128)


BN2 = TOK_PER_BATCH
BK2 = 4096
NI2 = NTOK // BN2
NJ2 = KCODES // BK2
INV_N = 1.0 / NTOK
INV_ELEMS = 1.0 / (TOK_PER_BATCH * EDIM)


def _enc_body(idx_ref, enc_ref, perp_ref, counts_s, ent_s):
    i = pl.program_id(0)
    j = pl.program_id(1)
    col0 = pl.multiple_of(j * BK2, BK2)

    idxv = idx_ref[...]
    iota = lax.broadcasted_iota(jnp.int32, (1, BK2), 1) + col0
    onehot = (iota == idxv).astype(jnp.float32)
    enc_ref[...] = onehot

    colsum = jnp.sum(onehot, axis=0, keepdims=True)

    @pl.when(i == 0)
    def _():
        counts_s[:, pl.ds(col0, BK2)] = colsum

    @pl.when(i > 0)
    def _():
        counts_s[:, pl.ds(col0, BK2)] += colsum

    @pl.when(i == NI2 - 1)
    def _():
        p = counts_s[:, pl.ds(col0, BK2)] * INV_N
        part = jnp.sum(p * jnp.log(p + 1e-10), axis=1, keepdims=True)

        @pl.when(j == 0)
        def _():
            ent_s[...] = part

        @pl.when(j > 0)
        def _():
            ent_s[...] += part

        @pl.when(j == NJ2 - 1)
        def _():
            perp_ref[...] = jnp.exp(-ent_s[...])


def _encodings_stats(idx_col):
    return pl.pallas_call(
        _enc_body,
        grid=(NI2, NJ2),
        in_specs=[
            pl.BlockSpec((BN2, 1), lambda i, j: (i, 0)),
        ],
        out_specs=[
            pl.BlockSpec((BN2, BK2), lambda i, j: (i, j)),
            pl.BlockSpec((1, 1), lambda i, j: (0, 0)),
        ],
        out_shape=[
            jax.ShapeDtypeStruct((NTOK, KCODES), jnp.float32),
            jax.ShapeDtypeStruct((1, 1), jnp.float32),
        ],
        scratch_shapes=[
            pltpu.VMEM((1, KCODES), jnp.float32),
            pltpu.VMEM((1, 1), jnp.float32),
        ],
    )(idx_col)


def _qst_body(x_ref, q_ref, qst_ref, loss_ref):
    x = x_ref[...]
    q = q_ref[:, :EDIM]
    t = q - x
    qst_ref[...] = x + t
    tsq = t * t
    parts = [jnp.sum(tsq[i * TOK_PER_BATCH:(i + 1) * TOK_PER_BATCH, :],
                     axis=(0, 1), keepdims=True) for i in range(NBATCH)]
    s = jnp.concatenate(parts, axis=0) * INV_ELEMS
    loss_ref[...] = s + CCOEF * s


def _qst_loss(flat, quant128):
    return pl.pallas_call(
        _qst_body,
        in_specs=[
            pl.BlockSpec((NTOK, EDIM), lambda: (0, 0)),
            pl.BlockSpec((NTOK, GATHER_W), lambda: (0, 0)),
        ],
        out_specs=[
            pl.BlockSpec((NTOK, EDIM), lambda: (0, 0)),
            pl.BlockSpec((NBATCH, 1), lambda: (0, 0)),
        ],
        out_shape=[
            jax.ShapeDtypeStruct((NTOK, EDIM), jnp.float32),
            jax.ShapeDtypeStruct((NBATCH, 1), jnp.float32),
        ],
    )(flat, quant128)


def kernel(inputs, W):
    x = jnp.transpose(inputs, (0, 2, 3, 1))
    flat = x.reshape(NTOK, EDIM)
    wt = W.T

    distances, idx_col = _distances_argmin(flat, wt)
    table128 = jnp.pad(W, ((0, 0), (0, GATHER_W - EDIM)))
    encodings, perp2 = _encodings_stats(idx_col)
    quant128 = _sc_gather(idx_col.reshape(NTOK), table128)
    qst, loss3 = _qst_loss(flat, quant128)

    quantized_out = jnp.transpose(qst.reshape(NBATCH, 32, 32, EDIM),
                                  (0, 3, 1, 2))
    loss = loss3.reshape(NBATCH)
    perplexity = perp2.reshape(())
    enc_idx = idx_col.reshape(NBATCH, TOK_PER_BATCH)
    return (quantized_out, loss, perplexity, encodings, enc_idx, distances)
